# baseline (speedup 1.0000x reference)
import functools

import jax
import jax.numpy as jnp
from jax.experimental import pallas as pl
from jax.experimental.pallas import tpu as pltpu

EXPERT_CAPACITY = 1280
_NSTREAMS = 16
_CHUNK = 256


def _router_body(*refs, blocks_per_batch, T, E):
    xs = refs[:_NSTREAMS]
    w_ref, logits_ref, eidx_ref, pmax_ref, carry_ref, triu_ref = \
        refs[_NSTREAMS:]
    C = _CHUNK
    i = pl.program_id(0)

    @pl.when(i == 0)
    def _():
        row = jax.lax.broadcasted_iota(jnp.int32, (C, C), 0)
        col = jax.lax.broadcasted_iota(jnp.int32, (C, C), 1)
        triu_ref[...] = (row <= col).astype(jnp.float32)

    @pl.when(i % blocks_per_batch == 0)
    def _():
        carry_ref[...] = jnp.zeros_like(carry_ref)

    dims = (((1,), (0,)), ((), ()))
    w = w_ref[...]
    lt = jnp.concatenate(
        [jax.lax.dot_general(x_ref[...], w, dims,
                             preferred_element_type=jnp.float32).T
         for x_ref in xs], axis=1)
    logits_ref[...] = lt

    m = jnp.max(lt, axis=0, keepdims=True)
    s = jnp.sum(jnp.exp(lt - m), axis=0, keepdims=True)
    pmax_ref[...] = (1.0 / s).reshape(1, 1, T)

    eiota = jax.lax.broadcasted_iota(jnp.int32, (E, T), 0)
    idx = jnp.min(jnp.where(lt == m, eiota, E), axis=0, keepdims=True)
    oh = (eiota == idx).astype(jnp.float32)

    triu = triu_ref[...]
    ps = [jax.lax.dot_general(oh[:, c * C:(c + 1) * C], triu, dims,
                              preferred_element_type=jnp.float32)
          for c in range(T // C)]
    off = carry_ref[...]
    pieces = []
    for p in ps:
        pieces.append(p + off)
        off = off + p[:, C - 1:C]
    prio = jnp.concatenate(pieces, axis=1)
    carry_ref[...] = off

    keep = (prio <= EXPERT_CAPACITY) & (oh > 0.0)
    eidx_ref[...] = keep.astype(jnp.int32)


def kernel(hidden_states, W):
    B, S, H = hidden_states.shape
    E = W.shape[0]
    T = 2048
    x = hidden_states.reshape(B * S, H)
    wT = W.T
    nblocks = (B * S) // T
    blocks_per_batch = S // T
    ns = _NSTREAMS
    Ts = T // ns

    def x_spec(k):
        return pl.BlockSpec((Ts, H), lambda i, k=k: (ns * i + k, 0))

    logits_t, eidx_t, pmax = pl.pallas_call(
        functools.partial(_router_body, blocks_per_batch=blocks_per_batch,
                          T=T, E=E),
        grid=(nblocks,),
        in_specs=[x_spec(k) for k in range(ns)] + [
            pl.BlockSpec((H, E), lambda i: (0, 0)),
        ],
        out_specs=[
            pl.BlockSpec((E, T), lambda i: (0, i)),
            pl.BlockSpec((E, T), lambda i: (0, i)),
            pl.BlockSpec((1, 1, T), lambda i: (i, 0, 0)),
        ],
        out_shape=[
            jax.ShapeDtypeStruct((E, B * S), jnp.float32),
            jax.ShapeDtypeStruct((E, B * S), jnp.int32),
            jax.ShapeDtypeStruct((nblocks, 1, T), jnp.float32),
        ],
        scratch_shapes=[pltpu.VMEM((E, 1), jnp.float32),
                        pltpu.VMEM((_CHUNK, _CHUNK), jnp.float32)],
        compiler_params=pltpu.CompilerParams(
            dimension_semantics=("arbitrary",)),
    )(*([x] * ns), wT)

    return (eidx_t.T.reshape(B, S, E),
            pmax.reshape(B, S, 1),
            logits_t.T.reshape(B, S, E))


# speedup vs baseline: 1.0086x; 1.0086x over previous
import functools

import jax
import jax.numpy as jnp
from jax.experimental import pallas as pl
from jax.experimental.pallas import tpu as pltpu

EXPERT_CAPACITY = 1280
_NSTREAMS = 16
_CHUNK = 256


def _router_body(*refs, blocks_per_batch, T, E):
    xs = refs[:_NSTREAMS]
    w_ref, logits_ref, eidx_ref, pmax_ref, carry_ref, triu_ref = \
        refs[_NSTREAMS:]
    C = _CHUNK
    i = pl.program_id(0)

    @pl.when(i == 0)
    def _():
        row = jax.lax.broadcasted_iota(jnp.int32, (C, C), 0)
        col = jax.lax.broadcasted_iota(jnp.int32, (C, C), 1)
        triu_ref[...] = (row <= col).astype(jnp.float32)

    @pl.when(i % blocks_per_batch == 0)
    def _():
        carry_ref[...] = jnp.zeros_like(carry_ref)

    dims = (((1,), (0,)), ((), ()))
    w = w_ref[...]
    lt = jnp.concatenate(
        [jax.lax.dot_general(x_ref[...], w, dims,
                             preferred_element_type=jnp.float32).T
         for x_ref in xs], axis=1)
    logits_ref[...] = lt

    m = jnp.max(lt, axis=0, keepdims=True)
    s = jnp.sum(jnp.exp(lt - m), axis=0, keepdims=True)
    pmax_ref[...] = (1.0 / s).reshape(1, 1, T)

    eiota = jax.lax.broadcasted_iota(jnp.int32, (E, T), 0)
    idx = jnp.min(jnp.where(lt == m, eiota, E), axis=0, keepdims=True)
    oh = (eiota == idx).astype(jnp.float32)

    triu = triu_ref[...]
    ps = [jax.lax.dot_general(oh[:, c * C:(c + 1) * C], triu, dims,
                              preferred_element_type=jnp.float32)
          for c in range(T // C)]
    off = carry_ref[...]
    pieces = []
    for p in ps:
        pieces.append(p + off)
        off = off + p[:, C - 1:C]
    prio = jnp.concatenate(pieces, axis=1)
    carry_ref[...] = off

    keep = (prio <= EXPERT_CAPACITY) & (oh > 0.0)
    eidx_ref[...] = keep.astype(jnp.int32)


def kernel(hidden_states, W):
    B, S, H = hidden_states.shape
    E = W.shape[0]
    T = 4096
    x = hidden_states.reshape(B * S, H)
    wT = W.T
    nblocks = (B * S) // T
    blocks_per_batch = S // T
    ns = _NSTREAMS
    Ts = T // ns

    def x_spec(k):
        return pl.BlockSpec((Ts, H), lambda i, k=k: (ns * i + k, 0))

    logits_t, eidx_t, pmax = pl.pallas_call(
        functools.partial(_router_body, blocks_per_batch=blocks_per_batch,
                          T=T, E=E),
        grid=(nblocks,),
        in_specs=[x_spec(k) for k in range(ns)] + [
            pl.BlockSpec((H, E), lambda i: (0, 0)),
        ],
        out_specs=[
            pl.BlockSpec((E, T), lambda i: (0, i)),
            pl.BlockSpec((E, T), lambda i: (0, i)),
            pl.BlockSpec((1, 1, T), lambda i: (i, 0, 0)),
        ],
        out_shape=[
            jax.ShapeDtypeStruct((E, B * S), jnp.float32),
            jax.ShapeDtypeStruct((E, B * S), jnp.int32),
            jax.ShapeDtypeStruct((nblocks, 1, T), jnp.float32),
        ],
        scratch_shapes=[pltpu.VMEM((E, 1), jnp.float32),
                        pltpu.VMEM((_CHUNK, _CHUNK), jnp.float32)],
        compiler_params=pltpu.CompilerParams(
            dimension_semantics=("arbitrary",)),
    )(*([x] * ns), wT)

    return (eidx_t.T.reshape(B, S, E),
            pmax.reshape(B, S, 1),
            logits_t.T.reshape(B, S, E))


# speedup vs baseline: 1.0546x; 1.0456x over previous
import functools

import jax
import jax.numpy as jnp
from jax.experimental import pallas as pl
from jax.experimental.pallas import tpu as pltpu

EXPERT_CAPACITY = 1280
_NSTREAMS = 16
_CHUNK = 256


def _router_body(*refs, blocks_per_batch, T, E):
    xs = refs[:_NSTREAMS]
    w_ref, logits_ref, eidx_ref, pmax_ref, carry_ref, triu_ref = \
        refs[_NSTREAMS:]
    C = _CHUNK
    i = pl.program_id(0)

    @pl.when(i == 0)
    def _():
        row = jax.lax.broadcasted_iota(jnp.int32, (C, C), 0)
        col = jax.lax.broadcasted_iota(jnp.int32, (C, C), 1)
        triu_ref[...] = (row <= col).astype(jnp.float32)

    @pl.when(i % blocks_per_batch == 0)
    def _():
        carry_ref[...] = jnp.zeros_like(carry_ref)

    dims = (((1,), (1,)), ((), ()))
    w = w_ref[...]
    lt = jnp.concatenate(
        [jax.lax.dot_general(x_ref[...], w, dims,
                             preferred_element_type=jnp.float32).T
         for x_ref in xs], axis=1)
    logits_ref[...] = lt

    m = jnp.max(lt, axis=0, keepdims=True)
    s = jnp.sum(jnp.exp(lt - m), axis=0, keepdims=True)
    pmax_ref[...] = 1.0 / s

    eiota = jax.lax.broadcasted_iota(jnp.int32, (E, T), 0)
    idx = jnp.min(jnp.where(lt == m, eiota, E), axis=0, keepdims=True)
    oh = (eiota == idx).astype(jnp.float32)

    triu = triu_ref[...]
    cdims = (((1,), (0,)), ((), ()))
    ps = [jax.lax.dot_general(oh[:, c * C:(c + 1) * C], triu, cdims,
                              preferred_element_type=jnp.float32)
          for c in range(T // C)]
    off = carry_ref[...]
    pieces = []
    for p in ps:
        pieces.append(p + off)
        off = off + p[:, C - 1:C]
    prio = jnp.concatenate(pieces, axis=1)
    carry_ref[...] = off

    keep = (prio <= EXPERT_CAPACITY) & (oh > 0.0)
    eidx_ref[...] = keep.astype(jnp.int32)


def kernel(hidden_states, W):
    B, S, H = hidden_states.shape
    E = W.shape[0]
    T = 4096
    x = hidden_states.reshape(B * S, H)
    nblocks = (B * S) // T
    blocks_per_batch = S // T
    ns = _NSTREAMS
    Ts = T // ns

    def x_spec(k):
        return pl.BlockSpec((Ts, H), lambda i, k=k: (ns * i + k, 0))

    logits_t, eidx_t, pmax = pl.pallas_call(
        functools.partial(_router_body, blocks_per_batch=blocks_per_batch,
                          T=T, E=E),
        grid=(nblocks,),
        in_specs=[x_spec(k) for k in range(ns)] + [
            pl.BlockSpec((E, H), lambda i: (0, 0)),
        ],
        out_specs=[
            pl.BlockSpec((E, T), lambda i: (0, i)),
            pl.BlockSpec((E, T), lambda i: (0, i)),
            pl.BlockSpec((1, T), lambda i: (0, i)),
        ],
        out_shape=[
            jax.ShapeDtypeStruct((E, B * S), jnp.float32),
            jax.ShapeDtypeStruct((E, B * S), jnp.int32),
            jax.ShapeDtypeStruct((1, B * S), jnp.float32),
        ],
        scratch_shapes=[pltpu.VMEM((E, 1), jnp.float32),
                        pltpu.VMEM((_CHUNK, _CHUNK), jnp.float32)],
        compiler_params=pltpu.CompilerParams(
            dimension_semantics=("arbitrary",)),
    )(*([x] * ns), W)

    return (eidx_t.T.reshape(B, S, E),
            pmax.reshape(B, S, 1),
            logits_t.T.reshape(B, S, E))


# speedup vs baseline: 1.0592x; 1.0043x over previous
import functools

import jax
import jax.numpy as jnp
from jax.experimental import pallas as pl
from jax.experimental.pallas import tpu as pltpu

EXPERT_CAPACITY = 1280
_NSTREAMS = 8
_CHUNK = 256


def _router_body(*refs, blocks_per_batch, T, E):
    xs = refs[:_NSTREAMS]
    w_ref, logits_ref, eidx_ref, pmax_ref, carry_ref, triu_ref = \
        refs[_NSTREAMS:]
    C = _CHUNK
    i = pl.program_id(0)

    @pl.when(i == 0)
    def _():
        row = jax.lax.broadcasted_iota(jnp.int32, (C, C), 0)
        col = jax.lax.broadcasted_iota(jnp.int32, (C, C), 1)
        triu_ref[...] = (row <= col).astype(jnp.float32)

    @pl.when(i % blocks_per_batch == 0)
    def _():
        carry_ref[...] = jnp.zeros_like(carry_ref)

    dims = (((1,), (1,)), ((), ()))
    w = w_ref[...]
    lt = jnp.concatenate(
        [jax.lax.dot_general(x_ref[...], w, dims,
                             preferred_element_type=jnp.float32).T
         for x_ref in xs], axis=1)
    logits_ref[...] = lt

    m = jnp.max(lt, axis=0, keepdims=True)
    s = jnp.sum(jnp.exp(lt - m), axis=0, keepdims=True)
    pmax_ref[...] = 1.0 / s

    eiota = jax.lax.broadcasted_iota(jnp.int32, (E, T), 0)
    idx = jnp.min(jnp.where(lt == m, eiota, E), axis=0, keepdims=True)
    oh = (eiota == idx).astype(jnp.float32)

    triu = triu_ref[...]
    cdims = (((1,), (0,)), ((), ()))
    ps = [jax.lax.dot_general(oh[:, c * C:(c + 1) * C], triu, cdims,
                              preferred_element_type=jnp.float32)
          for c in range(T // C)]
    off = carry_ref[...]
    pieces = []
    for p in ps:
        pieces.append(p + off)
        off = off + p[:, C - 1:C]
    prio = jnp.concatenate(pieces, axis=1)
    carry_ref[...] = off

    keep = (prio <= EXPERT_CAPACITY) & (oh > 0.0)
    eidx_ref[...] = keep.astype(jnp.int32)


def kernel(hidden_states, W):
    B, S, H = hidden_states.shape
    E = W.shape[0]
    T = 4096
    x = hidden_states.reshape(B * S, H)
    nblocks = (B * S) // T
    blocks_per_batch = S // T
    ns = _NSTREAMS
    Ts = T // ns

    def x_spec(k):
        return pl.BlockSpec((Ts, H), lambda i, k=k: (ns * i + k, 0))

    logits_t, eidx_t, pmax = pl.pallas_call(
        functools.partial(_router_body, blocks_per_batch=blocks_per_batch,
                          T=T, E=E),
        grid=(nblocks,),
        in_specs=[x_spec(k) for k in range(ns)] + [
            pl.BlockSpec((E, H), lambda i: (0, 0)),
        ],
        out_specs=[
            pl.BlockSpec((E, T), lambda i: (0, i)),
            pl.BlockSpec((E, T), lambda i: (0, i)),
            pl.BlockSpec((1, T), lambda i: (0, i)),
        ],
        out_shape=[
            jax.ShapeDtypeStruct((E, B * S), jnp.float32),
            jax.ShapeDtypeStruct((E, B * S), jnp.int32),
            jax.ShapeDtypeStruct((1, B * S), jnp.float32),
        ],
        scratch_shapes=[pltpu.VMEM((E, 1), jnp.float32),
                        pltpu.VMEM((_CHUNK, _CHUNK), jnp.float32)],
        compiler_params=pltpu.CompilerParams(
            dimension_semantics=("arbitrary",)),
    )(*([x] * ns), W)

    return (eidx_t.T.reshape(B, S, E),
            pmax.reshape(B, S, 1),
            logits_t.T.reshape(B, S, E))
